# Initial kernel scaffold; baseline (speedup 1.0000x reference)
import jax
import jax.numpy as jnp
from jax.experimental import pallas as pl


def kernel(x, router_logits, w13_weight, w2_weight):
    raise NotImplementedError("write your pallas kernel here")


# speedup vs baseline: 1.8577x; 1.8577x over previous
import functools

import jax
import jax.numpy as jnp
from jax.experimental import pallas as pl
from jax.experimental.pallas import tpu as pltpu

E = 8
TOPK = 2
TB = 512


def _moe_body(x_ref, rl_ref, w13_ref, w2_ref, out_ref):
    e = pl.program_id(0)
    t = pl.program_id(1)
    rows = pl.ds(t * TB, TB)

    xs = x_ref[rows, :]
    w13 = w13_ref[0]
    h = jax.lax.dot_general(xs, w13, (((1,), (1,)), ((), ())),
                            preferred_element_type=jnp.float32)
    ff = h.shape[1] // 2
    gate = h[:, :ff]
    up = h[:, ff:]
    act = gate * (1.0 / (1.0 + jnp.exp(-gate))) * up
    y = jax.lax.dot_general(act, w2_ref[0], (((1,), (1,)), ((), ())),
                            preferred_element_type=jnp.float32)

    l = rl_ref[rows, :]
    p = jnp.exp(l - jnp.max(l, axis=1, keepdims=True))
    pn = p / jnp.sum(p, axis=1, keepdims=True)
    iota = jax.lax.broadcasted_iota(jnp.int32, pn.shape, 1)
    m1 = jnp.max(pn, axis=1, keepdims=True)
    i1 = jnp.min(jnp.where(pn == m1, iota, E), axis=1, keepdims=True)
    pn2 = jnp.where(iota == i1, -1.0, pn)
    m2 = jnp.max(pn2, axis=1, keepdims=True)
    i2 = jnp.min(jnp.where(pn2 == m2, iota, E), axis=1, keepdims=True)
    sel = (iota == i1) | (iota == i2)
    comb = jnp.where(sel, pn, 0.0) / (m1 + m2)
    c = jnp.sum(comb * (iota == e).astype(jnp.float32), axis=1, keepdims=True)

    czy = c * y

    @pl.when(e == 0)
    def _():
        out_ref[rows, :] = czy

    @pl.when(e > 0)
    def _():
        out_ref[rows, :] = out_ref[rows, :] + czy


@jax.jit
def kernel(x, router_logits, w13_weight, w2_weight):
    T, D = x.shape
    n_t = T // TB
    return pl.pallas_call(
        _moe_body,
        grid=(E, n_t),
        in_specs=[
            pl.BlockSpec((T, D), lambda e, t: (0, 0)),
            pl.BlockSpec((T, E), lambda e, t: (0, 0)),
            pl.BlockSpec((1, 2 * w13_weight.shape[1] // 2, D), lambda e, t: (e, 0, 0)),
            pl.BlockSpec((1, D, w2_weight.shape[2]), lambda e, t: (e, 0, 0)),
        ],
        out_specs=pl.BlockSpec((T, D), lambda e, t: (0, 0)),
        out_shape=jax.ShapeDtypeStruct((T, D), jnp.float32),
        compiler_params=pltpu.CompilerParams(
            dimension_semantics=("arbitrary", "arbitrary"),
        ),
    )(x, router_logits, w13_weight, w2_weight)
